# Initial kernel scaffold; baseline (speedup 1.0000x reference)
import jax
import jax.numpy as jnp
from jax.experimental import pallas as pl


def kernel(word_inputs, sent_tokens, word_seq_lengths, seq_token_masks, embedding, lstm_w_ih, lstm_b, lstm_w_hh_f, lstm_w_hh_b, sent_lstm_w_ih, sent_lstm_b, sent_lstm_w_hh_f, sent_lstm_w_hh_b, w_gate_f, w_gate_s, b_gate):
    raise NotImplementedError("write your pallas kernel here")


# speedup vs baseline: 1.1334x; 1.1334x over previous
import functools

import jax
import jax.numpy as jnp
from jax import lax
from jax.experimental import pallas as pl
from jax.experimental.pallas import tpu as pltpu


def _cell(gates, c_prev, H):
    i_g = jax.nn.sigmoid(gates[:, 0:H])
    f_g = jax.nn.sigmoid(gates[:, H:2 * H])
    g_g = jnp.tanh(gates[:, 2 * H:3 * H])
    o_g = jax.nn.sigmoid(gates[:, 3 * H:4 * H])
    c_new = f_g * c_prev + i_g * g_g
    h_new = o_g * jnp.tanh(c_new)
    return h_new, c_new


def _bilstm_pair_kernel(xw_ref, xs_ref, len_ref,
                        wih_ref, wb_ref, whh_ref,
                        sih_ref, sb_ref, swhh_ref,
                        outw_ref, outs_ref, gx_ref,
                        *, T, B, L, BS, H):
    i = pl.program_id(0)
    sign = 1 - 2 * i

    gx_ref[0:L * BS, :] = jnp.dot(
        xs_ref[...], sih_ref[0], preferred_element_type=jnp.float32
    ) + sb_ref[0]
    swhh = swhh_ref[0]
    base_s = i * (L - 1)

    def sstep(l, carry):
        h, c = carry
        l_eff = base_s + sign * l
        row = pl.multiple_of(l_eff * BS, BS)
        gates = gx_ref[pl.ds(row, BS), :] + jnp.dot(
            h.astype(jnp.bfloat16), swhh, preferred_element_type=jnp.float32)
        h, c = _cell(gates, c, H)
        outs_ref[l_eff] = h
        return h, c

    zs = jnp.zeros((BS, H), jnp.float32)
    lax.fori_loop(0, L, sstep, (zs, zs), unroll=True)

    gx_ref[0:T * B, :] = jnp.dot(
        xw_ref[...], wih_ref[0], preferred_element_type=jnp.float32
    ) + wb_ref[0]
    whh = whh_ref[0]
    lens = len_ref[...]
    base_w = i * (T - 1)

    def wstep(t, carry):
        h, c = carry
        t_eff = base_w + sign * t
        row = pl.multiple_of(t_eff * B, B)
        gates = gx_ref[pl.ds(row, B), :] + jnp.dot(
            h.astype(jnp.bfloat16), whh, preferred_element_type=jnp.float32)
        h_cand, c_cand = _cell(gates, c, H)
        valid = t_eff < lens
        outw_ref[t_eff] = jnp.where(valid, h_cand, 0.0)
        h = jnp.where(valid, h_cand, h)
        c = jnp.where(valid, c_cand, c)
        return h, c

    zw = jnp.zeros((B, H), jnp.float32)
    lax.fori_loop(0, T, wstep, (zw, zw), unroll=True)


def _blend_kernel(f_ref, s_ref, wf_ref, ws_ref, b_ref, o_ref):
    f = f_ref[...]
    s = s_ref[...]
    logits = (
        jnp.dot(f.astype(jnp.bfloat16), wf_ref[...],
                preferred_element_type=jnp.float32)
        + jnp.dot(s.astype(jnp.bfloat16), ws_ref[...],
                  preferred_element_type=jnp.float32)
        + b_ref[...]
    )
    gamma = jax.nn.sigmoid(logits)
    o_ref[...] = gamma * f + (1.0 - gamma) * s


def kernel(word_inputs, sent_tokens, word_seq_lengths, seq_token_masks,
           embedding, lstm_w_ih, lstm_b, lstm_w_hh_f, lstm_w_hh_b,
           sent_lstm_w_ih, sent_lstm_b, sent_lstm_w_hh_f, sent_lstm_w_hh_b,
           w_gate_f, w_gate_s, b_gate):
    B, T = word_inputs.shape
    _, S, L = sent_tokens.shape
    D = embedding.shape[1]
    H = lstm_w_hh_f.shape[0]
    BS = B * S
    two_h = 2 * H

    xw = embedding[jnp.transpose(word_inputs).reshape(-1)]
    xw = xw.astype(jnp.bfloat16)
    xs = embedding[jnp.transpose(sent_tokens, (2, 0, 1)).reshape(-1)]
    xs = xs.astype(jnp.bfloat16)
    lens = word_seq_lengths.astype(jnp.int32).reshape(B, 1)

    wih = lstm_w_ih.reshape(D, 2, 4 * H).transpose(1, 0, 2)
    wb = lstm_b.reshape(1, 2, 4 * H).transpose(1, 0, 2)
    whh = jnp.stack([lstm_w_hh_f, lstm_w_hh_b])
    sih = sent_lstm_w_ih.reshape(D, 2, 4 * H).transpose(1, 0, 2)
    sb = sent_lstm_b.reshape(1, 2, 4 * H).transpose(1, 0, 2)
    swhh = jnp.stack([sent_lstm_w_hh_f, sent_lstm_w_hh_b])

    gx_rows = max(T * B, L * BS)
    outw, outs = pl.pallas_call(
        functools.partial(_bilstm_pair_kernel, T=T, B=B, L=L, BS=BS, H=H),
        out_shape=[jax.ShapeDtypeStruct((T, B, two_h), jnp.float32),
                   jax.ShapeDtypeStruct((L, BS, two_h), jnp.float32)],
        grid=(2,),
        in_specs=[
            pl.BlockSpec((T * B, D), lambda i: (0, 0)),
            pl.BlockSpec((L * BS, D), lambda i: (0, 0)),
            pl.BlockSpec((B, 1), lambda i: (0, 0)),
            pl.BlockSpec((1, D, 4 * H), lambda i: (i, 0, 0)),
            pl.BlockSpec((1, 1, 4 * H), lambda i: (i, 0, 0)),
            pl.BlockSpec((1, H, 4 * H), lambda i: (i, 0, 0)),
            pl.BlockSpec((1, D, 4 * H), lambda i: (i, 0, 0)),
            pl.BlockSpec((1, 1, 4 * H), lambda i: (i, 0, 0)),
            pl.BlockSpec((1, H, 4 * H), lambda i: (i, 0, 0)),
        ],
        out_specs=[
            pl.BlockSpec((T, B, H), lambda i: (0, 0, i)),
            pl.BlockSpec((L, BS, H), lambda i: (0, 0, i)),
        ],
        scratch_shapes=[pltpu.VMEM((gx_rows, 4 * H), jnp.float32)],
        compiler_params=pltpu.CompilerParams(
            dimension_semantics=("parallel",)),
    )(xw, xs, lens, wih, wb, whh, sih, sb, swhh)

    feature_out = jnp.transpose(outw, (1, 0, 2))
    sent_btc = outs.reshape(L, B, S, two_h).transpose(1, 2, 0, 3)
    sent_btc = sent_btc.reshape(B, S * L, two_h)
    if S * L == T:
        fos = sent_btc
    elif S * L > T:
        fos = sent_btc[:, :T, :]
    else:
        fos = jnp.zeros((B, T, two_h), jnp.float32)
        fos = fos.at[:, :S * L, :].set(sent_btc)

    N = B * T
    f2 = feature_out.reshape(N, two_h)
    s2 = fos.reshape(N, two_h)
    bn = N if N <= 1024 else 1024
    out = pl.pallas_call(
        _blend_kernel,
        out_shape=jax.ShapeDtypeStruct((N, two_h), jnp.float32),
        grid=(pl.cdiv(N, bn),),
        in_specs=[
            pl.BlockSpec((bn, two_h), lambda i: (i, 0)),
            pl.BlockSpec((bn, two_h), lambda i: (i, 0)),
            pl.BlockSpec((two_h, two_h), lambda i: (0, 0)),
            pl.BlockSpec((two_h, two_h), lambda i: (0, 0)),
            pl.BlockSpec((1, two_h), lambda i: (0, 0)),
        ],
        out_specs=pl.BlockSpec((bn, two_h), lambda i: (i, 0)),
        compiler_params=pltpu.CompilerParams(
            dimension_semantics=("parallel",)),
    )(f2, s2, w_gate_f, w_gate_s, b_gate)
    return out.reshape(B, T, two_h)
